# Initial kernel scaffold; baseline (speedup 1.0000x reference)
import functools

import jax
import jax.numpy as jnp
from jax import lax
from jax.experimental import pallas as pl
from jax.experimental.pallas import tpu as pltpu
from jax.experimental.pallas import tpu_sc as plsc

NUM_EMOTIONS = 1000
EMO_DIM = 64
BATCH = 16384

_NC = 2
_NS = 16
_L = 16
_NW = _NC * _NS
_BPW = BATCH // _NW

_mesh = plsc.VectorSubcoreMesh(core_axis_name="c", subcore_axis_name="s")


@jax.jit
def _emotion_encode(emo_id, strength, table):
    @functools.partial(
        pl.kernel,
        out_type=jax.ShapeDtypeStruct((BATCH, EMO_DIM), jnp.float32),
        mesh=_mesh,
        scratch_types=[
            pltpu.VMEM((_BPW,), jnp.int32),
            pltpu.VMEM((_BPW,), jnp.float32),
            pltpu.VMEM((_BPW, EMO_DIM), jnp.float32),
            pltpu.SemaphoreType.DMA,
        ],
    )
    def k(emo_hbm, str_hbm, table_hbm, out_hbm, idx_v, str_v, rows_v, sem):
        wid = lax.axis_index("s") * _NC + lax.axis_index("c")
        base = wid * _BPW
        pltpu.sync_copy(emo_hbm.at[pl.ds(base, _BPW)], idx_v)
        pltpu.sync_copy(str_hbm.at[pl.ds(base, _BPW)], str_v)
        pltpu.async_copy(table_hbm.at[idx_v], rows_v, sem).wait()

        @pl.loop(0, _BPW)
        def _(b):
            bvec = jnp.broadcast_to(b, (_L,)).astype(jnp.int32)
            s = plsc.load_gather(str_v, [bvec])
            for j in range(EMO_DIM // _L):
                sl = pl.ds(j * _L, _L)
                rows_v[b, sl] = rows_v[b, sl] * s

        pltpu.sync_copy(rows_v, out_hbm.at[pl.ds(base, _BPW)])

    return k(emo_id, strength, table)


def kernel(emo_id, strength, table):
    return _emotion_encode(emo_id.astype(jnp.int32), strength, table)


# speedup vs baseline: 1.4716x; 1.4716x over previous
import dataclasses
import functools

import jax
import jax.numpy as jnp
from jax import lax
from jax.experimental import pallas as pl
from jax.experimental.pallas import tpu as pltpu
from jax.experimental.pallas import tpu_sc as plsc

NUM_EMOTIONS = 1000
EMO_DIM = 64
BATCH = 16384

_NC = 2
_NS = 16
_L = 16
_NW = _NC * _NS
_BPW = BATCH // _NW

_mesh = plsc.VectorSubcoreMesh(core_axis_name="c", subcore_axis_name="s")

_cp = pltpu.CompilerParams()
if "needs_layout_passes" in pltpu.CompilerParams.__dataclass_fields__:
    _cp = dataclasses.replace(_cp, needs_layout_passes=False)
if "use_tc_tiling_on_sc" in pltpu.CompilerParams.__dataclass_fields__:
    _cp = dataclasses.replace(_cp, use_tc_tiling_on_sc=False)


@jax.jit
def _emotion_encode(emo_id, strength, table):
    @functools.partial(
        pl.kernel,
        out_type=jax.ShapeDtypeStruct((BATCH, EMO_DIM), jnp.float32),
        mesh=_mesh,
        compiler_params=_cp,
        scratch_types=[
            pltpu.VMEM((_BPW,), jnp.int32),
            pltpu.VMEM((_BPW,), jnp.float32),
            pltpu.VMEM((_BPW, EMO_DIM), jnp.float32),
            pltpu.SemaphoreType.DMA,
        ],
    )
    def k(emo_hbm, str_hbm, table_hbm, out_hbm, idx_v, str_v, rows_v, sem):
        wid = lax.axis_index("s") * _NC + lax.axis_index("c")
        base = wid * _BPW
        pltpu.sync_copy(emo_hbm.at[pl.ds(base, _BPW)], idx_v)
        pltpu.sync_copy(str_hbm.at[pl.ds(base, _BPW)], str_v)
        pltpu.async_copy(table_hbm.at[idx_v], rows_v, sem).wait()

        @pl.loop(0, _BPW)
        def _(b):
            bvec = jnp.broadcast_to(b, (_L,)).astype(jnp.int32)
            s = plsc.load_gather(str_v, [bvec])
            for j in range(EMO_DIM // _L):
                sl = pl.ds(j * _L, _L)
                rows_v[b, sl] = rows_v[b, sl] * s

        pltpu.sync_copy(rows_v, out_hbm.at[pl.ds(base, _BPW)])

    return k(emo_id, strength, table)


def kernel(emo_id, strength, table):
    return _emotion_encode(emo_id.astype(jnp.int32), strength, table)
